# Initial kernel scaffold; baseline (speedup 1.0000x reference)
import jax
import jax.numpy as jnp
from jax.experimental import pallas as pl


def kernel(elements, batch, ptr, bag, d_mean_trans, d_log_stds, orientation_template):
    raise NotImplementedError("write your pallas kernel here")


# speedup vs baseline: 233.1759x; 233.1759x over previous
import functools

import jax
import jax.numpy as jnp
import numpy as np
from jax import lax
from jax.experimental import pallas as pl
from jax.experimental.pallas import tpu as pltpu
from jax.experimental.pallas import tpu_sc as plsc

MIN_D, MAX_D = 0.95, 1.8
_D_CENTER = (MIN_D + MAX_D) / 2.0
_D_HALF_WIDTH = (MAX_D - MIN_D) / 2.0
_LOG_EPS = float(np.log(1e-6))
_HALF_LOG_2PI = float(0.5 * np.log(2.0 * np.pi))

_NUM_CORES = 2
_NUM_SUBCORES = 16
_NW = _NUM_CORES * _NUM_SUBCORES
_L = 16


@functools.lru_cache(maxsize=None)
def _build_sc_call(B: int, E: int):
    ch = -(-B // _NW)
    ch = -(-ch // _L) * _L
    assert ch % 8 == 0 and B % 8 == 0 and ch <= B and (_NW - 1) * ch <= B
    groups = ch // _L

    mesh = plsc.VectorSubcoreMesh(
        core_axis_name="c", subcore_axis_name="s",
        num_cores=_NUM_CORES, num_subcores=_NUM_SUBCORES)

    @functools.partial(
        pl.kernel,
        out_type=(
            jax.ShapeDtypeStruct((B,), jnp.int32),
            jax.ShapeDtypeStruct((B,), jnp.int32),
            jax.ShapeDtypeStruct((B,), jnp.float32),
            jax.ShapeDtypeStruct((3 * B,), jnp.float32),
            jax.ShapeDtypeStruct((B,), jnp.float32),
        ),
        mesh=mesh,
        compiler_params=pltpu.CompilerParams(needs_layout_passes=False),
        scratch_types=[
            pltpu.VMEM((ch * E,), jnp.int32),
            pltpu.VMEM((ch,), jnp.int32),
            pltpu.VMEM((ch,), jnp.int32),
            pltpu.VMEM((ch,), jnp.float32),
            pltpu.VMEM((3 * ch,), jnp.float32),
            pltpu.VMEM((ch,), jnp.float32),
            pltpu.VMEM((5 * _L,), jnp.float32),
        ],
    )
    def sc_call(params_hbm, bag_hbm, focus_hbm, elem_hbm, dist_hbm,
                orient_hbm, logp_hbm, bag_v, focus_v, elem_v, dist_v,
                orient_v, logp_v, params_v):
        wid = lax.axis_index("s") * _NUM_CORES + lax.axis_index("c")
        base = jnp.minimum(wid * ch, B - ch)
        base = pl.multiple_of(base, 8)

        pltpu.sync_copy(params_hbm, params_v)
        pltpu.sync_copy(bag_hbm.at[pl.ds(base * E, ch * E)], bag_v)

        d_raw = params_v[pl.ds(0, _L)]
        log_std = params_v[pl.ds(_L, _L)]
        pat0 = params_v[pl.ds(2 * _L, _L)]
        pat1 = params_v[pl.ds(3 * _L, _L)]
        pat2 = params_v[pl.ds(4 * _L, _L)]

        e2x = jnp.exp(d_raw * 2.0)
        d_mean = (1.0 - 2.0 / (e2x + 1.0)) * _D_HALF_WIDTH + _D_CENTER
        logp_vec = -jnp.maximum(log_std, _LOG_EPS) - _HALF_LOG_2PI
        zero_vec = jnp.zeros((_L,), jnp.int32)
        lane = lax.iota(jnp.int32, _L)

        def body(g, _):
            r0 = g * _L
            word0 = (r0 + lane) * E
            elem = zero_vec
            for j in range(E - 1, -1, -1):
                v = plsc.load_gather(bag_v, [word0 + j])
                elem = jnp.where(v > 0, j, elem)
            elem_v[pl.ds(r0, _L)] = elem
            focus_v[pl.ds(r0, _L)] = zero_vec
            dist_v[pl.ds(r0, _L)] = d_mean
            logp_v[pl.ds(r0, _L)] = logp_vec
            o0 = 3 * r0
            orient_v[pl.ds(o0, _L)] = pat0
            orient_v[pl.ds(o0 + _L, _L)] = pat1
            orient_v[pl.ds(o0 + 2 * _L, _L)] = pat2
            return _

        lax.fori_loop(0, groups, body, None)

        pltpu.sync_copy(focus_v, focus_hbm.at[pl.ds(base, ch)])
        pltpu.sync_copy(elem_v, elem_hbm.at[pl.ds(base, ch)])
        pltpu.sync_copy(dist_v, dist_hbm.at[pl.ds(base, ch)])
        pltpu.sync_copy(orient_v, orient_hbm.at[pl.ds(3 * base, 3 * ch)])
        pltpu.sync_copy(logp_v, logp_hbm.at[pl.ds(base, ch)])

    return sc_call


def kernel(elements, batch, ptr, bag, d_mean_trans, d_log_stds,
           orientation_template):
    B = ptr.shape[0] - 1
    E = bag.shape[1]
    bag = bag.astype(jnp.int32).reshape(-1)
    params = jnp.concatenate([
        jnp.broadcast_to(d_mean_trans.reshape(-1)[:1].astype(jnp.float32), (_L,)),
        jnp.broadcast_to(d_log_stds.reshape(-1)[:1].astype(jnp.float32), (_L,)),
        jnp.tile(orientation_template.reshape(-1).astype(jnp.float32), (_L,)),
    ])
    focus, element, dist, orient, logp = _build_sc_call(B, E)(params, bag)
    return (focus, element, dist.reshape(B, 1), orient.reshape(B, 3),
            logp.reshape(B, 1))
